# Initial kernel scaffold; baseline (speedup 1.0000x reference)
import jax
import jax.numpy as jnp
from jax.experimental import pallas as pl


def kernel(x):
    raise NotImplementedError("write your pallas kernel here")


# speedup vs baseline: 2.5155x; 2.5155x over previous
import functools

import jax
import jax.numpy as jnp
from jax import lax
from jax.experimental import pallas as pl

NBINS = 256
R_MIN = 0.0
R_MAX = 255.0
_SPLIT = 16


def _hist_body(x_ref, w_ref, o_ref, *, sub, nsub):
    j = pl.program_id(1)

    @pl.when(j == 0)
    def _init():
        o_ref[...] = jnp.zeros_like(o_ref)

    scale = NBINS / (R_MAX - R_MIN)
    iota_hi = lax.broadcasted_iota(jnp.int32, (1, _SPLIT), 1)
    iota_lo = lax.broadcasted_iota(jnp.int32, (1, _SPLIT), 1)

    def step(t, acc):
        xs = x_ref[0, 0, pl.ds(t * sub, sub)]
        ws = w_ref[0, 0, pl.ds(t * sub, sub)]
        idx = jnp.floor((xs - R_MIN) * scale).astype(jnp.int32)
        idx = jnp.clip(idx, 0, NBINS - 1)
        in_range = (xs >= R_MIN) & (xs <= R_MAX)
        w_eff = jnp.where(in_range, ws, 0.0)
        hi = idx // _SPLIT
        lo = idx - hi * _SPLIT
        a = (hi[:, None] == iota_hi).astype(jnp.float32)
        b = jnp.where(lo[:, None] == iota_lo, w_eff[:, None], 0.0)
        return acc + lax.dot_general(
            a, b, (((0,), (0,)), ((), ())), preferred_element_type=jnp.float32
        )

    acc = lax.fori_loop(
        0, nsub, step, jnp.zeros((_SPLIT, _SPLIT), jnp.float32)
    )
    o_ref[0] += acc


def kernel(x):
    b, c, h, w = x.shape
    n = h * w
    rows = b * c
    weight = (1.0 / n) * jnp.zeros((1, 1, n), dtype=jnp.float32)
    flat = x.reshape(rows, 1, n)

    sub = min(n, 8192)
    chunk = min(n, 8 * sub)
    assert n % chunk == 0 and chunk % sub == 0
    nchunks = n // chunk
    nsub = chunk // sub

    out = pl.pallas_call(
        functools.partial(_hist_body, sub=sub, nsub=nsub),
        grid=(rows, nchunks),
        in_specs=[
            pl.BlockSpec((1, 1, chunk), lambda i, j: (i, 0, j)),
            pl.BlockSpec((1, 1, chunk), lambda i, j: (0, 0, j)),
        ],
        out_specs=pl.BlockSpec((1, _SPLIT, _SPLIT), lambda i, j: (i, 0, 0)),
        out_shape=jax.ShapeDtypeStruct((rows, _SPLIT, _SPLIT), jnp.float32),
    )(flat, weight)
    return out.reshape(b, c, NBINS)


# speedup vs baseline: 38.2336x; 15.1992x over previous
import functools

import jax
import jax.numpy as jnp
from jax import lax
from jax.experimental import pallas as pl
from jax.experimental.pallas import tpu as pltpu
from jax.experimental.pallas import tpu_sc as plsc

NBINS = 256
R_MIN = 0.0
R_MAX = 255.0
_LANES = 16
_CHUNK = 16384


def _make_sc_hist(rows, n):
    info = plsc.get_sparse_core_info()
    nc, ns = info.num_cores, info.num_subcores
    assert rows == nc * ns, (rows, nc, ns)
    assert n % _CHUNK == 0
    nchunks = n // _CHUNK
    scale = NBINS / (R_MAX - R_MIN)

    mesh = plsc.VectorSubcoreMesh(core_axis_name="c", subcore_axis_name="s")

    @functools.partial(
        pl.kernel,
        mesh=mesh,
        out_type=jax.ShapeDtypeStruct((rows, NBINS), jnp.float32),
        scratch_types=[
            pltpu.VMEM((_CHUNK,), jnp.float32),
            pltpu.VMEM((_CHUNK,), jnp.float32),
            pltpu.VMEM((NBINS,), jnp.float32),
        ],
        compiler_params=pltpu.CompilerParams(needs_layout_passes=False),
    )
    def run(x_hbm, w_hbm, out_hbm, xbuf, wbuf, hist):
        wid = lax.axis_index("s") * nc + lax.axis_index("c")

        for i in range(NBINS // _LANES):
            hist[pl.ds(i * _LANES, _LANES)] = jnp.zeros((_LANES,), jnp.float32)

        def chunk_body(ci, _):
            pltpu.sync_copy(x_hbm.at[wid, pl.ds(ci * _CHUNK, _CHUNK)], xbuf)
            pltpu.sync_copy(w_hbm.at[pl.ds(ci * _CHUNK, _CHUNK)], wbuf)

            def vec_body(i, _):
                off = i * _LANES
                xv = xbuf[pl.ds(off, _LANES)]
                wv = wbuf[pl.ds(off, _LANES)]
                idx = ((xv - R_MIN) * scale).astype(jnp.int32)
                idx = jnp.clip(idx, 0, NBINS - 1)
                ok = (xv >= R_MIN) & (xv <= R_MAX)
                weff = jnp.where(ok, wv, jnp.zeros((_LANES,), jnp.float32))
                plsc.addupdate_scatter(hist, [idx], weff)
                return 0

            lax.fori_loop(0, _CHUNK // _LANES, vec_body, 0)
            return 0

        lax.fori_loop(0, nchunks, chunk_body, 0)
        pltpu.sync_copy(hist, out_hbm.at[wid])

    return run


def kernel(x):
    b, c, h, w = x.shape
    n = h * w
    rows = b * c
    weight = (1.0 / n) * jnp.zeros((n,), dtype=jnp.float32)
    flat = x.reshape(rows, n)
    out = _make_sc_hist(rows, n)(flat, weight)
    return out.reshape(b, c, NBINS)


# speedup vs baseline: 42.5759x; 1.1136x over previous
import functools

import jax
import jax.numpy as jnp
from jax import lax
from jax.experimental import pallas as pl
from jax.experimental.pallas import tpu as pltpu
from jax.experimental.pallas import tpu_sc as plsc

NBINS = 256
R_MIN = 0.0
R_MAX = 255.0
_LANES = 16
_CHUNK = 16384
_NBUF = 2


def _make_sc_hist(rows, n):
    info = plsc.get_sparse_core_info()
    nc, ns = info.num_cores, info.num_subcores
    assert rows == nc * ns, (rows, nc, ns)
    assert n % (_CHUNK * _NBUF) == 0
    nchunks = n // _CHUNK
    scale = NBINS / (R_MAX - R_MIN)

    mesh = plsc.VectorSubcoreMesh(core_axis_name="c", subcore_axis_name="s")

    @functools.partial(
        pl.kernel,
        mesh=mesh,
        out_type=jax.ShapeDtypeStruct((rows, NBINS), jnp.float32),
        scratch_types=[
            pltpu.VMEM((_NBUF, _CHUNK), jnp.float32),
            pltpu.VMEM((_NBUF, _CHUNK), jnp.float32),
            pltpu.VMEM((NBINS,), jnp.float32),
            pltpu.SemaphoreType.DMA,
            pltpu.SemaphoreType.DMA,
        ],
        compiler_params=pltpu.CompilerParams(needs_layout_passes=False),
    )
    def run(x_hbm, w_hbm, out_hbm, xbuf, wbuf, hist, semx, semw):
        wid = lax.axis_index("s") * nc + lax.axis_index("c")

        for i in range(NBINS // _LANES):
            hist[pl.ds(i * _LANES, _LANES)] = jnp.zeros((_LANES,), jnp.float32)

        def start(ci, slot):
            pltpu.async_copy(
                x_hbm.at[wid, pl.ds(ci * _CHUNK, _CHUNK)], xbuf.at[slot], semx
            )
            pltpu.async_copy(
                w_hbm.at[pl.ds(ci * _CHUNK, _CHUNK)], wbuf.at[slot], semw
            )

        def drain(slot):
            pltpu.make_async_copy(
                x_hbm.at[wid, pl.ds(0, _CHUNK)], xbuf.at[slot], semx
            ).wait()
            pltpu.make_async_copy(
                w_hbm.at[pl.ds(0, _CHUNK)], wbuf.at[slot], semw
            ).wait()

        for slot in range(_NBUF):
            start(slot, slot)

        @pl.loop(0, nchunks, step=_NBUF)
        def chunk_pair(g):
            for slot in range(_NBUF):
                ci = g + slot
                drain(slot)

                @pl.loop(0, _CHUNK // _LANES, unroll=8)
                def vec_body(i):
                    off = i * _LANES
                    xv = xbuf[slot, pl.ds(off, _LANES)]
                    wv = wbuf[slot, pl.ds(off, _LANES)]
                    idx = ((xv - R_MIN) * scale).astype(jnp.int32)
                    idx = jnp.clip(idx, 0, NBINS - 1)
                    ok = (xv >= R_MIN) & (xv <= R_MAX)
                    weff = jnp.where(ok, wv, jnp.zeros((_LANES,), jnp.float32))
                    plsc.addupdate_scatter(hist, [idx], weff)

                @pl.when(ci + _NBUF < nchunks)
                def _next():
                    start(ci + _NBUF, slot)

        pltpu.sync_copy(hist, out_hbm.at[wid])

    return run


def kernel(x):
    b, c, h, w = x.shape
    n = h * w
    rows = b * c
    weight = (1.0 / n) * jnp.zeros((n,), dtype=jnp.float32)
    flat = x.reshape(rows, n)
    out = _make_sc_hist(rows, n)(flat, weight)
    return out.reshape(b, c, NBINS)


# speedup vs baseline: 98.1160x; 2.3045x over previous
import functools

import jax
import jax.numpy as jnp
from jax import lax
from jax.experimental import pallas as pl
from jax.experimental.pallas import tpu as pltpu
from jax.experimental.pallas import tpu_sc as plsc

NBINS = 256
R_MIN = 0.0
R_MAX = 255.0
_LANES = 16
_CHUNK = 16384
_NBUF = 2


def _make_sc_hist(rows, n):
    info = plsc.get_sparse_core_info()
    nc, ns = info.num_cores, info.num_subcores
    assert rows == nc * ns, (rows, nc, ns)
    assert n % (_CHUNK * _NBUF) == 0
    nchunks = n // _CHUNK
    scale = NBINS / (R_MAX - R_MIN)

    mesh = plsc.VectorSubcoreMesh(core_axis_name="c", subcore_axis_name="s")

    @functools.partial(
        pl.kernel,
        mesh=mesh,
        out_type=jax.ShapeDtypeStruct((rows, NBINS), jnp.float32),
        scratch_types=[
            pltpu.VMEM((_NBUF, _CHUNK), jnp.float32),
            pltpu.VMEM((_NBUF, _CHUNK), jnp.float32),
            pltpu.VMEM((NBINS,), jnp.float32),
            pltpu.SemaphoreType.DMA,
            pltpu.SemaphoreType.DMA,
        ],
        compiler_params=pltpu.CompilerParams(needs_layout_passes=False),
    )
    def run(x_hbm, w_hbm, out_hbm, xbuf, wbuf, hist, semx, semw):
        wid = lax.axis_index("s") * nc + lax.axis_index("c")

        for i in range(NBINS // _LANES):
            hist[pl.ds(i * _LANES, _LANES)] = jnp.zeros((_LANES,), jnp.float32)

        def start(ci, slot):
            pltpu.async_copy(
                x_hbm.at[wid, pl.ds(ci * _CHUNK, _CHUNK)], xbuf.at[slot], semx
            )
            pltpu.async_copy(
                w_hbm.at[pl.ds(ci * _CHUNK, _CHUNK)], wbuf.at[slot], semw
            )

        def drain(slot):
            pltpu.make_async_copy(
                x_hbm.at[wid, pl.ds(0, _CHUNK)], xbuf.at[slot], semx
            ).wait()
            pltpu.make_async_copy(
                w_hbm.at[pl.ds(0, _CHUNK)], wbuf.at[slot], semw
            ).wait()

        for slot in range(_NBUF):
            start(slot, slot)

        @pl.loop(0, nchunks, step=_NBUF)
        def chunk_pair(g):
            for slot in range(_NBUF):
                ci = g + slot
                drain(slot)

                @plsc.parallel_loop(0, _CHUNK // _LANES, unroll=8)
                def vec_body(i):
                    off = i * _LANES
                    xv = xbuf[slot, pl.ds(off, _LANES)]
                    wv = wbuf[slot, pl.ds(off, _LANES)]
                    idx = ((xv - R_MIN) * scale).astype(jnp.int32)
                    idx = jnp.clip(idx, 0, NBINS - 1)
                    ok = (xv >= R_MIN) & (xv <= R_MAX)
                    weff = jnp.where(ok, wv, jnp.zeros((_LANES,), jnp.float32))
                    plsc.addupdate_scatter(hist, [idx], weff)

                @pl.when(ci + _NBUF < nchunks)
                def _next():
                    start(ci + _NBUF, slot)

        pltpu.sync_copy(hist, out_hbm.at[wid])

    return run


def kernel(x):
    b, c, h, w = x.shape
    n = h * w
    rows = b * c
    weight = (1.0 / n) * jnp.zeros((n,), dtype=jnp.float32)
    flat = x.reshape(rows, n)
    out = _make_sc_hist(rows, n)(flat, weight)
    return out.reshape(b, c, NBINS)
